# baseline (speedup 1.0000x reference)
import functools

import jax
import jax.numpy as jnp
from jax import lax
from jax.experimental import pallas as pl
from jax.experimental.pallas import tpu as pltpu
from jax.experimental.pallas import tpu_sc as plsc

_NC = 2
_NS = 16
_NW = _NC * _NS
_TCH = 512
_CHUNK = 1280


def _mesh():
    return plsc.VectorSubcoreMesh(core_axis_name="c", subcore_axis_name="s")


@functools.partial(jax.jit, static_argnames=("vocab", "dim"))
def _compact_table(table_t, table_tail, vocab, dim):
    rows_per_w = (vocab // _NW) // 32 * 32
    tail = vocab - rows_per_w * _NW
    n_chunks = rows_per_w // _TCH
    group = 128 // dim

    @functools.partial(
        pl.kernel,
        mesh=_mesh(),
        out_type=jax.ShapeDtypeStruct((vocab // group, 128), jnp.float32),
        compiler_params=pltpu.CompilerParams(needs_layout_passes=False),
        scratch_types=[
            pltpu.VMEM((dim, _TCH), jnp.float32),
            pltpu.VMEM((dim, _TCH), jnp.float32),
            pltpu.VMEM((_TCH // group, 128), jnp.float32),
            pltpu.VMEM((_TCH // group, 128), jnp.float32),
            pltpu.VMEM((64, 32), jnp.float32),
            pltpu.SemaphoreType.DMA,
            pltpu.SemaphoreType.DMA,
            pltpu.SemaphoreType.DMA,
            pltpu.SemaphoreType.DMA,
        ],
    )
    def k(tt_hbm, tail_hbm, tc_hbm, in0, in1, ot0, ot1, tailv,
          gi0, gi1, go0, go1):
        wid = lax.axis_index("s") * _NC + lax.axis_index("c")
        base = wid * rows_per_w

        def i_copy(r0, n, buf, sem):
            return pltpu.make_async_copy(
                tt_hbm.at[:, pl.ds(pl.multiple_of(r0, 128), n)],
                buf.at[:, pl.ds(0, n)], sem)

        def o_copy(r0, n, buf, sem):
            return pltpu.make_async_copy(
                buf.at[pl.ds(0, n // group)],
                tc_hbm.at[pl.ds(pl.multiple_of(r0 // group, 8), n // group)],
                sem)

        lanes = lax.iota(jnp.int32, 16)
        idx_lo = lanes
        idx_hi = lanes + 16

        def transpose(buf, obuf, n):
            def mstep(m8, carry):
                m0 = m8 * 8
                base = jnp.full((16,), group * m0, jnp.int32)
                for dm in range(8):
                    for g in range(group):
                        col = base + (group * dm + g)
                        obuf[m0 + dm, pl.ds(g * dim, 16)] = (
                            plsc.load_gather(buf, [idx_lo, col]))
                        obuf[m0 + dm, pl.ds(g * dim + 16, 16)] = (
                            plsc.load_gather(buf, [idx_hi, col]))
                return carry
            lax.fori_loop(0, n // group // 8, mstep, 0)

        i_copy(base, _TCH, in0, gi0).start()

        def pair(p, carry):
            ra = base + (2 * p) * _TCH
            rb = ra + _TCH

            i_copy(ra, _TCH, in0, gi0).wait()

            @pl.when(2 * p + 1 < n_chunks)
            def _():
                i_copy(rb, _TCH, in1, gi1).start()

            @pl.when(p > 0)
            def _():
                o_copy(ra - 2 * _TCH, _TCH, ot0, go0).wait()
            transpose(in0, ot0, _TCH)
            o_copy(ra, _TCH, ot0, go0).start()

            @pl.when(2 * p + 1 < n_chunks)
            def _():
                i_copy(rb, _TCH, in1, gi1).wait()

                @pl.when(2 * p + 2 < n_chunks)
                def _():
                    i_copy(rb + _TCH, _TCH, in0, gi0).start()

                @pl.when(p > 0)
                def _():
                    o_copy(rb - 2 * _TCH, _TCH, ot1, go1).wait()
                transpose(in1, ot1, _TCH)
                o_copy(rb, _TCH, ot1, go1).start()
            return carry

        n_pairs = (n_chunks + 1) // 2
        lax.fori_loop(0, n_pairs, pair, 0)
        o_copy(base + (n_chunks - 2) * _TCH, _TCH, ot1, go1).wait()
        o_copy(base + (n_chunks - 1) * _TCH, _TCH, ot0, go0).wait()

        @pl.when(wid == 0)
        def _():
            t0 = rows_per_w * _NW
            i_copy(t0, _TCH, in0, gi0).start()
            pltpu.make_async_copy(tail_hbm, tailv, gi1).start()
            i_copy(t0, _TCH, in0, gi0).wait()
            transpose(in0, ot0, _TCH)
            o_copy(t0, _TCH, ot0, go0).start()
            pltpu.make_async_copy(tail_hbm, tailv, gi1).wait()

            def tstep(m, carry):
                for g in range(group):
                    for c in range(dim // 16):
                        ot1[m, pl.ds(g * dim + c * 16, 16)] = (
                            tailv[group * m + g, pl.ds(c * 16, 16)])
                return carry
            lax.fori_loop(0, 64 // group, tstep, 0)
            o_copy(t0 + _TCH, 64, ot1, go1).start()
            o_copy(t0, _TCH, ot0, go0).wait()
            o_copy(t0 + _TCH, 64, ot1, go1).wait()

    return k(table_t, table_tail)


@functools.partial(jax.jit, static_argnames=("total_b", "dim"))
def _gather_rows(idx, table, total_b, dim):
    b_per_w = total_b // _NW
    n_chunks = b_per_w // _CHUNK
    n_pairs = n_chunks // 2

    @functools.partial(
        pl.kernel,
        mesh=_mesh(),
        out_type=jax.ShapeDtypeStruct((total_b, dim), jnp.float32),
        compiler_params=pltpu.CompilerParams(use_tc_tiling_on_sc=False),
        scratch_types=[
            pltpu.VMEM((b_per_w,), jnp.int32),
            pltpu.VMEM((_CHUNK, dim), jnp.float32),
            pltpu.VMEM((_CHUNK, dim), jnp.float32),
            pltpu.SemaphoreType.DMA,
            pltpu.SemaphoreType.DMA,
            pltpu.SemaphoreType.DMA,
            pltpu.SemaphoreType.DMA,
        ],
    )
    def k(idx_hbm, table_hbm, out_hbm, idx_v, rows0, rows1, gs0, gs1, os0, os1):
        wid = lax.axis_index("s") * _NC + lax.axis_index("c")
        base = wid * b_per_w
        pltpu.sync_copy(idx_hbm.at[pl.ds(base, b_per_w)], idx_v)

        def g_copy(c, buf, sem):
            return pltpu.make_async_copy(
                table_hbm.at[idx_v.at[pl.ds(c * _CHUNK, _CHUNK)]], buf, sem)

        def o_copy(c, buf, sem):
            return pltpu.make_async_copy(
                buf, out_hbm.at[pl.ds(base + c * _CHUNK, _CHUNK)], sem)

        g_copy(0, rows0, gs0).start()

        def body(p, carry):
            ce = 2 * p
            co = ce + 1

            @pl.when(p > 0)
            def _():
                o_copy(co - 2, rows1, os1).wait()

            g_copy(co, rows1, gs1).start()
            g_copy(ce, rows0, gs0).wait()
            o_copy(ce, rows0, os0).start()
            g_copy(co, rows1, gs1).wait()
            o_copy(ce, rows0, os0).wait()

            @pl.when(p < n_pairs - 1)
            def _():
                g_copy(ce + 2, rows0, gs0).start()

            o_copy(co, rows1, os1).start()
            return carry

        lax.fori_loop(0, n_pairs, body, 0)
        o_copy(n_chunks - 1, rows1, os1).wait()

    return k(idx, table)


def kernel(xs, table):
    b, t = xs.shape
    v, dim = table.shape
    tail_rows = (v // _NW) // 32 * 32 * _NW + _TCH
    tc = _compact_table(table.T, table[tail_rows:], vocab=v, dim=dim)
    tflat = tc.reshape(v, dim)
    idx = xs.reshape(-1).astype(jnp.int32)
    out = _gather_rows(idx, tflat, total_b=b * t, dim=dim)
    return out.reshape(b, t, dim)


# speedup vs baseline: 1.2115x; 1.2115x over previous
import functools

import jax
import jax.numpy as jnp
from jax import lax
from jax.experimental import pallas as pl
from jax.experimental.pallas import tpu as pltpu
from jax.experimental.pallas import tpu_sc as plsc

_NC = 2
_NS = 16
_NW = _NC * _NS
_TCH = 256
_CHUNK = 1280


def _mesh():
    return plsc.VectorSubcoreMesh(core_axis_name="c", subcore_axis_name="s")


@functools.partial(jax.jit, static_argnames=("vocab", "dim"))
def _compact_table(table, vocab, dim):
    rows_per_w = (vocab // _NW) // 32 * 32
    tail = vocab - rows_per_w * _NW
    n_chunks = rows_per_w // _TCH
    group = 128 // dim

    @functools.partial(
        pl.kernel,
        mesh=_mesh(),
        out_type=jax.ShapeDtypeStruct((vocab // group, 128), jnp.float32),
        compiler_params=pltpu.CompilerParams(needs_layout_passes=False),
        scratch_types=[
            pltpu.VMEM((_TCH, dim), jnp.float32),
            pltpu.VMEM((_TCH, dim), jnp.float32),
            pltpu.VMEM((_TCH // group, 128), jnp.float32),
            pltpu.VMEM((_TCH // group, 128), jnp.float32),
            pltpu.SemaphoreType.DMA,
            pltpu.SemaphoreType.DMA,
            pltpu.SemaphoreType.DMA,
            pltpu.SemaphoreType.DMA,
        ],
    )
    def k(t_hbm, tc_hbm, in0, in1, ot0, ot1, gi0, gi1, go0, go1):
        wid = lax.axis_index("s") * _NC + lax.axis_index("c")
        base = wid * rows_per_w

        def i_copy(r0, n, buf, sem):
            return pltpu.make_async_copy(
                t_hbm.at[pl.ds(pl.multiple_of(r0, 8), n)],
                buf.at[pl.ds(0, n)], sem)

        def o_copy(r0, n, buf, sem):
            return pltpu.make_async_copy(
                buf.at[pl.ds(0, n // group)],
                tc_hbm.at[pl.ds(pl.multiple_of(r0 // group, 8), n // group)],
                sem)

        def transpose(buf, obuf, n):
            def mstep(m8, carry):
                m0 = m8 * 8
                for dm in range(8):
                    for g in range(group):
                        for c in range(dim // 16):
                            obuf[m0 + dm, pl.ds(g * dim + c * 16, 16)] = (
                                buf[group * (m0 + dm) + g, pl.ds(c * 16, 16)])
                return carry
            lax.fori_loop(0, n // group // 8, mstep, 0)

        i_copy(base, _TCH, in0, gi0).start()

        def pair(p, carry):
            ra = base + (2 * p) * _TCH
            rb = ra + _TCH

            i_copy(ra, _TCH, in0, gi0).wait()

            @pl.when(2 * p + 1 < n_chunks)
            def _():
                i_copy(rb, _TCH, in1, gi1).start()

            @pl.when(p > 0)
            def _():
                o_copy(ra - 2 * _TCH, _TCH, ot0, go0).wait()
            transpose(in0, ot0, _TCH)
            o_copy(ra, _TCH, ot0, go0).start()

            @pl.when(2 * p + 1 < n_chunks)
            def _():
                i_copy(rb, _TCH, in1, gi1).wait()

                @pl.when(2 * p + 2 < n_chunks)
                def _():
                    i_copy(rb + _TCH, _TCH, in0, gi0).start()

                @pl.when(p > 0)
                def _():
                    o_copy(rb - 2 * _TCH, _TCH, ot1, go1).wait()
                transpose(in1, ot1, _TCH)
                o_copy(rb, _TCH, ot1, go1).start()
            return carry

        n_pairs = (n_chunks + 1) // 2
        lax.fori_loop(0, n_pairs, pair, 0)
        last0, last1 = ((n_chunks - 1, n_chunks - 2) if n_chunks % 2
                        else (n_chunks - 2, n_chunks - 1))
        o_copy(base + last1 * _TCH, _TCH, ot1, go1).wait()
        o_copy(base + last0 * _TCH, _TCH, ot0, go0).wait()

        @pl.when(wid == 0)
        def _():
            t0 = rows_per_w * _NW
            i_copy(t0, _TCH, in0, gi0).start()
            i_copy(t0 + _TCH, _TCH, in1, gi1).start()
            i_copy(t0, _TCH, in0, gi0).wait()
            transpose(in0, ot0, _TCH)
            o_copy(t0, _TCH, ot0, go0).start()
            i_copy(t0 + _TCH, _TCH, in1, gi1).wait()
            transpose(in1, ot1, _TCH)
            o_copy(t0 + _TCH, _TCH, ot1, go1).start()
            i_copy(t0 + 2 * _TCH, 64, in0, gi0).start()
            i_copy(t0 + 2 * _TCH, 64, in0, gi0).wait()
            o_copy(t0, _TCH, ot0, go0).wait()
            transpose(in0, ot0, 64)
            o_copy(t0 + 2 * _TCH, 64, ot0, go0).start()
            o_copy(t0 + _TCH, _TCH, ot1, go1).wait()
            o_copy(t0 + 2 * _TCH, 64, ot0, go0).wait()

    return k(table)


@functools.partial(jax.jit, static_argnames=("total_b", "dim"))
def _gather_rows(idx, table, total_b, dim):
    b_per_w = total_b // _NW
    n_chunks = b_per_w // _CHUNK
    n_pairs = n_chunks // 2

    @functools.partial(
        pl.kernel,
        mesh=_mesh(),
        out_type=jax.ShapeDtypeStruct((total_b, dim), jnp.float32),
        compiler_params=pltpu.CompilerParams(use_tc_tiling_on_sc=False),
        scratch_types=[
            pltpu.VMEM((b_per_w,), jnp.int32),
            pltpu.VMEM((_CHUNK, dim), jnp.float32),
            pltpu.VMEM((_CHUNK, dim), jnp.float32),
            pltpu.SemaphoreType.DMA,
            pltpu.SemaphoreType.DMA,
            pltpu.SemaphoreType.DMA,
            pltpu.SemaphoreType.DMA,
        ],
    )
    def k(idx_hbm, table_hbm, out_hbm, idx_v, rows0, rows1, gs0, gs1, os0, os1):
        wid = lax.axis_index("s") * _NC + lax.axis_index("c")
        base = wid * b_per_w
        pltpu.sync_copy(idx_hbm.at[pl.ds(base, b_per_w)], idx_v)

        def g_copy(c, buf, sem):
            return pltpu.make_async_copy(
                table_hbm.at[idx_v.at[pl.ds(c * _CHUNK, _CHUNK)]], buf, sem)

        def o_copy(c, buf, sem):
            return pltpu.make_async_copy(
                buf, out_hbm.at[pl.ds(base + c * _CHUNK, _CHUNK)], sem)

        g_copy(0, rows0, gs0).start()

        def body(p, carry):
            ce = 2 * p
            co = ce + 1

            @pl.when(p > 0)
            def _():
                o_copy(co - 2, rows1, os1).wait()

            g_copy(co, rows1, gs1).start()
            g_copy(ce, rows0, gs0).wait()
            o_copy(ce, rows0, os0).start()
            g_copy(co, rows1, gs1).wait()
            o_copy(ce, rows0, os0).wait()

            @pl.when(p < n_pairs - 1)
            def _():
                g_copy(ce + 2, rows0, gs0).start()

            o_copy(co, rows1, os1).start()
            return carry

        lax.fori_loop(0, n_pairs, body, 0)
        o_copy(n_chunks - 1, rows1, os1).wait()

    return k(idx, table)


def kernel(xs, table):
    b, t = xs.shape
    v, dim = table.shape
    tc = _compact_table(table, vocab=v, dim=dim)
    tflat = tc.reshape(v, dim)
    idx = xs.reshape(-1).astype(jnp.int32)
    out = _gather_rows(idx, tflat, total_b=b * t, dim=dim)
    return out.reshape(b, t, dim)


# speedup vs baseline: 1.2872x; 1.0625x over previous
import functools

import jax
import jax.numpy as jnp
from jax import lax
from jax.experimental import pallas as pl
from jax.experimental.pallas import tpu as pltpu
from jax.experimental.pallas import tpu_sc as plsc

_NC = 2
_NS = 16
_NW = _NC * _NS
_R = 4


@functools.partial(jax.jit, static_argnames=("batch", "hist", "dim"))
def _embed(xs, table, batch, hist, dim):
    rows_per_w = batch // _NW
    n_groups = rows_per_w // _R
    n_pairs = n_groups // 2
    mesh = plsc.VectorSubcoreMesh(core_axis_name="c", subcore_axis_name="s")

    @functools.partial(
        pl.kernel,
        mesh=mesh,
        out_type=jax.ShapeDtypeStruct((batch, hist, dim), jnp.float32),
        compiler_params=pltpu.CompilerParams(use_tc_tiling_on_sc=False),
        scratch_types=[
            pltpu.VMEM((rows_per_w, hist), jnp.int32),
            pltpu.VMEM((_R, hist, dim), jnp.float32),
            pltpu.VMEM((_R, hist, dim), jnp.float32),
            pltpu.SemaphoreType.DMA,
            pltpu.SemaphoreType.DMA,
            pltpu.SemaphoreType.DMA,
            pltpu.SemaphoreType.DMA,
        ],
    )
    def k(xs_hbm, table_hbm, out_hbm, idx_v, buf0, buf1, gs0, gs1, os0, os1):
        wid = lax.axis_index("s") * _NC + lax.axis_index("c")
        base = wid * rows_per_w
        pltpu.sync_copy(xs_hbm.at[pl.ds(base, rows_per_w)], idx_v)

        def g_copy(g, j, buf, sem):
            return pltpu.make_async_copy(
                table_hbm.at[idx_v.at[g * _R + j]], buf.at[j], sem)

        def o_copy(g, buf, sem):
            return pltpu.make_async_copy(
                buf, out_hbm.at[pl.ds(base + g * _R, _R)], sem)

        for j in range(_R):
            g_copy(0, j, buf0, gs0).start()

        def body(p, carry):
            ge = 2 * p
            go = ge + 1

            @pl.when(p > 0)
            def _():
                o_copy(go - 2, buf1, os1).wait()

            for j in range(_R):
                g_copy(go, j, buf1, gs1).start()
            for j in range(_R):
                g_copy(ge, j, buf0, gs0).wait()
            o_copy(ge, buf0, os0).start()
            for j in range(_R):
                g_copy(go, j, buf1, gs1).wait()
            o_copy(ge, buf0, os0).wait()

            @pl.when(p < n_pairs - 1)
            def _():
                for j in range(_R):
                    g_copy(ge + 2, j, buf0, gs0).start()

            o_copy(go, buf1, os1).start()
            return carry

        lax.fori_loop(0, n_pairs, body, 0)
        o_copy(n_groups - 1, buf1, os1).wait()

    return k(xs, table)


def kernel(xs, table):
    b, t = xs.shape
    dim = table.shape[1]
    return _embed(xs.astype(jnp.int32), table, batch=b, hist=t, dim=dim)
